# baseline (speedup 1.0000x reference)
import jax
import jax.numpy as jnp
from jax import lax
from jax.experimental import pallas as pl
from jax.experimental.pallas import tpu as pltpu
from jax.experimental.pallas import tpu_sc as plsc

_B = 16384
_EDIM = 8
_GRP = 128 // _EDIM
_NC = 2
_NS = 16
_NW = _NC * _NS
_BPW = _B // _NW

_BLK = 2048
_H0 = 64
_H1 = 32


def _gather3(ugrp, igrp, pgrp, user_t, item_t, pub_t):
    mesh = plsc.VectorSubcoreMesh(core_axis_name="c", subcore_axis_name="s")
    out_t = jax.ShapeDtypeStruct((_B, 128), jnp.float32)

    @pl.kernel(
        out_type=(out_t, out_t, out_t),
        mesh=mesh,
        scratch_types=[
            pltpu.VMEM((_BPW,), jnp.int32),
            pltpu.VMEM((_BPW,), jnp.int32),
            pltpu.VMEM((_BPW,), jnp.int32),
            pltpu.VMEM((_BPW, 128), jnp.float32),
            pltpu.SemaphoreType.DMA,
        ],
    )
    def gather_kernel(uidx_hbm, iidx_hbm, pidx_hbm,
                      utab_hbm, itab_hbm, ptab_hbm,
                      uout_hbm, iout_hbm, pout_hbm,
                      uidx_v, iidx_v, pidx_v, rows_v, sem):
        wid = lax.axis_index("s") * _NC + lax.axis_index("c")
        sl = pl.ds(wid * _BPW, _BPW)
        pltpu.sync_copy(uidx_hbm.at[sl], uidx_v)
        pltpu.sync_copy(iidx_hbm.at[sl], iidx_v)
        pltpu.sync_copy(pidx_hbm.at[sl], pidx_v)
        pltpu.async_copy(utab_hbm.at[uidx_v], rows_v, sem).wait()
        pltpu.sync_copy(rows_v, uout_hbm.at[sl])
        pltpu.async_copy(itab_hbm.at[iidx_v], rows_v, sem).wait()
        pltpu.sync_copy(rows_v, iout_hbm.at[sl])
        pltpu.async_copy(ptab_hbm.at[pidx_v], rows_v, sem).wait()
        pltpu.sync_copy(rows_v, pout_hbm.at[sl])

    def compact(t):
        piece = t.shape[0] // _GRP
        return jnp.concatenate(
            [lax.slice_in_dim(t, k * piece, (k + 1) * piece) for k in
             range(_GRP)], axis=1)

    return gather_kernel(ugrp, igrp, pgrp,
                         compact(user_t), compact(item_t), compact(pub_t))


def _mlp_body(ue2, ie2, pe2, sel3, pr, ft,
              w0u, w0i, w0p, w0pr, w0f, b0, w1, b1, w2, b2, out):
    lane_grp = lax.broadcasted_iota(jnp.int32, (_BLK, 128), 1) >> 3
    xu = jnp.where(lane_grp == sel3[:, 0:1], ue2[...], 0.0)
    xi = jnp.where(lane_grp == sel3[:, 1:2], ie2[...], 0.0)
    xp = jnp.where(lane_grp == sel3[:, 2:3], pe2[...], 0.0)
    acc = jnp.dot(xu, w0u[...], preferred_element_type=jnp.float32)
    acc += jnp.dot(xi, w0i[...], preferred_element_type=jnp.float32)
    acc += jnp.dot(xp, w0p[...], preferred_element_type=jnp.float32)
    acc += jnp.dot(ft[...], w0f[...], preferred_element_type=jnp.float32)
    acc += pr[...] * w0pr[...]
    h0 = jnp.maximum(acc + b0[...], 0.0)
    h1 = jnp.maximum(
        jnp.dot(h0, w1[...], preferred_element_type=jnp.float32) + b1[...], 0.0)
    out[...] = jax.nn.sigmoid(
        jnp.dot(h1, w2[...], preferred_element_type=jnp.float32) + b2[...])


def _mlp(ue2, ie2, pe2, sel3, price, features, W0, b0, W1, b1, W2, b2):
    W0t = W0.T
    w0u = jnp.tile(W0t[0:_EDIM], (_GRP, 1))
    w0i = jnp.tile(W0t[_EDIM:2 * _EDIM], (_GRP, 1))
    w0p = jnp.tile(W0t[2 * _EDIM:3 * _EDIM], (_GRP, 1))
    w0pr = W0t[3 * _EDIM:3 * _EDIM + 1]
    w0f = W0t[3 * _EDIM + 1:]
    nblk = _B // _BLK
    batch_spec = lambda w: pl.BlockSpec((_BLK, w), lambda i: (i, 0))
    full_spec = lambda a: pl.BlockSpec(a.shape, lambda i: (0,) * a.ndim)
    weights = (w0u, w0i, w0p, w0pr, w0f, b0.reshape(1, -1),
               W1.T, b1.reshape(1, -1), W2.T, b2.reshape(1, -1))
    return pl.pallas_call(
        _mlp_body,
        grid=(nblk,),
        in_specs=[batch_spec(128), batch_spec(128), batch_spec(128),
                  batch_spec(3), batch_spec(1), batch_spec(16)]
                 + [full_spec(w) for w in weights],
        out_specs=pl.BlockSpec((_BLK, 1), lambda i: (i, 0)),
        out_shape=jax.ShapeDtypeStruct((_B, 1), jnp.float32),
    )(ue2, ie2, pe2, sel3, price.reshape(_B, 1), features, *weights)


def kernel(user_input, item_input, publisher_input, price, features,
           user_table, item_table, pub_table,
           W0, b0, W1, b1, W2, b2):
    pu = user_table.shape[0] // _GRP
    pi = item_table.shape[0] // _GRP
    pp = pub_table.shape[0] // _GRP
    ue2, ie2, pe2 = _gather3(user_input % pu, item_input % pi,
                             publisher_input % pp,
                             user_table, item_table, pub_table)
    sel3 = jnp.stack([user_input // pu, item_input // pi,
                      publisher_input // pp], axis=1)
    return _mlp(ue2, ie2, pe2, sel3, price, features, W0, b0, W1, b1, W2, b2)


# speedup vs baseline: 1.4601x; 1.4601x over previous
import jax
import jax.numpy as jnp
from jax import lax
from jax.experimental import pallas as pl
from jax.experimental.pallas import tpu as pltpu
from jax.experimental.pallas import tpu_sc as plsc

_B = 16384
_EDIM = 8
_GRP = 128 // _EDIM
_NC = 2
_NS = 16
_NW = _NC * _NS
_BPW = _B // _NW

_BLK = 2048
_H0 = 64
_H1 = 32


def _gather1(grp_idx, tab128):
    mesh = plsc.VectorSubcoreMesh(core_axis_name="c", subcore_axis_name="s")

    @pl.kernel(
        out_type=jax.ShapeDtypeStruct((_B, 128), jnp.float32),
        mesh=mesh,
        scratch_types=[
            pltpu.VMEM((_BPW,), jnp.int32),
            pltpu.VMEM((_BPW, 128), jnp.float32),
            pltpu.SemaphoreType.DMA,
        ],
    )
    def gather_kernel(idx_hbm, tab_hbm, out_hbm, idx_v, rows_v, sem):
        wid = lax.axis_index("s") * _NC + lax.axis_index("c")
        sl = pl.ds(wid * _BPW, _BPW)
        pltpu.sync_copy(idx_hbm.at[sl], idx_v)
        pltpu.async_copy(tab_hbm.at[idx_v], rows_v, sem).wait()
        pltpu.sync_copy(rows_v, out_hbm.at[sl])

    return gather_kernel(grp_idx, tab128)


def _gather3(ugrp, igrp, pgrp, user_t, item_t, pub_t):
    ue2 = _gather1(ugrp, user_t.reshape(-1, 128))
    ie2 = _gather1(igrp, item_t.reshape(-1, 128))
    pe2 = _gather1(pgrp, pub_t.reshape(-1, 128))
    return ue2, ie2, pe2


def _mlp_body(ue2, ie2, pe2, sel3, pr, ft,
              w0u, w0i, w0p, w0pr, w0f, b0, w1, b1, w2, b2, out):
    lane_grp = lax.broadcasted_iota(jnp.int32, (_BLK, 128), 1) >> 3
    xu = jnp.where(lane_grp == sel3[:, 0:1], ue2[...], 0.0)
    xi = jnp.where(lane_grp == sel3[:, 1:2], ie2[...], 0.0)
    xp = jnp.where(lane_grp == sel3[:, 2:3], pe2[...], 0.0)
    acc = jnp.dot(xu, w0u[...], preferred_element_type=jnp.float32)
    acc += jnp.dot(xi, w0i[...], preferred_element_type=jnp.float32)
    acc += jnp.dot(xp, w0p[...], preferred_element_type=jnp.float32)
    acc += jnp.dot(ft[...], w0f[...], preferred_element_type=jnp.float32)
    acc += pr[...] * w0pr[...]
    h0 = jnp.maximum(acc + b0[...], 0.0)
    h1 = jnp.maximum(
        jnp.dot(h0, w1[...], preferred_element_type=jnp.float32) + b1[...], 0.0)
    out[...] = jax.nn.sigmoid(
        jnp.dot(h1, w2[...], preferred_element_type=jnp.float32) + b2[...])


def _mlp(ue2, ie2, pe2, sel3, price, features, W0, b0, W1, b1, W2, b2):
    W0t = W0.T
    w0u = jnp.tile(W0t[0:_EDIM], (_GRP, 1))
    w0i = jnp.tile(W0t[_EDIM:2 * _EDIM], (_GRP, 1))
    w0p = jnp.tile(W0t[2 * _EDIM:3 * _EDIM], (_GRP, 1))
    w0pr = W0t[3 * _EDIM:3 * _EDIM + 1]
    w0f = W0t[3 * _EDIM + 1:]
    nblk = _B // _BLK
    batch_spec = lambda w: pl.BlockSpec((_BLK, w), lambda i: (i, 0))
    full_spec = lambda a: pl.BlockSpec(a.shape, lambda i: (0,) * a.ndim)
    weights = (w0u, w0i, w0p, w0pr, w0f, b0.reshape(1, -1),
               W1.T, b1.reshape(1, -1), W2.T, b2.reshape(1, -1))
    return pl.pallas_call(
        _mlp_body,
        grid=(nblk,),
        in_specs=[batch_spec(128), batch_spec(128), batch_spec(128),
                  batch_spec(3), batch_spec(1), batch_spec(16)]
                 + [full_spec(w) for w in weights],
        out_specs=pl.BlockSpec((_BLK, 1), lambda i: (i, 0)),
        out_shape=jax.ShapeDtypeStruct((_B, 1), jnp.float32),
    )(ue2, ie2, pe2, sel3, price.reshape(_B, 1), features, *weights)


def kernel(user_input, item_input, publisher_input, price, features,
           user_table, item_table, pub_table,
           W0, b0, W1, b1, W2, b2):
    ue2, ie2, pe2 = _gather3(user_input >> 4, item_input >> 4,
                             publisher_input >> 4,
                             user_table, item_table, pub_table)
    sel3 = jnp.stack([user_input & (_GRP - 1), item_input & (_GRP - 1),
                      publisher_input & (_GRP - 1)], axis=1)
    return _mlp(ue2, ie2, pe2, sel3, price, features, W0, b0, W1, b1, W2, b2)
